# Initial kernel scaffold; baseline (speedup 1.0000x reference)
import jax
import jax.numpy as jnp
from jax import lax
from jax.experimental import pallas as pl
from jax.experimental.pallas import tpu as pltpu
from jax.experimental.pallas import tpu_sc as plsc

_N = 10000
_E = 320000
_NW = 32
_CH = 80
_JG = 2 * _E // (_NW * _CH)
_JS = _E // (_NW * _CH)
_NPT = _N // 16

_BN = 2000
_BE = 4000


def _gather_body(table, idx, out, idx_v, rows_v, sem):
    cid = lax.axis_index("c")
    sid = lax.axis_index("s")
    wid = sid * 2 + cid
    pltpu.sync_copy(idx.at[wid], idx_v)

    def step(j, carry):
        pltpu.async_copy(table.at[idx_v.at[j]], rows_v, sem).wait()
        pltpu.sync_copy(rows_v, out.at[wid].at[j])
        return carry

    lax.fori_loop(0, _JG, step, 0)


_gather = pl.kernel(
    _gather_body,
    out_type=jax.ShapeDtypeStruct((_NW, _JG, _CH, 32), jnp.float32),
    mesh=plsc.VectorSubcoreMesh(core_axis_name="c", subcore_axis_name="s"),
    scratch_types=[
        pltpu.VMEM((_JG, _CH), jnp.int32),
        pltpu.VMEM((_CH, 32), jnp.float32),
        pltpu.SemaphoreType.DMA,
    ],
)


def _scatter_body(m, dstw, zero, out, idx_v, m_v, acc):
    cid = lax.axis_index("c")
    sid = lax.axis_index("s")
    wid = sid * 2 + cid
    pltpu.sync_copy(dstw.at[wid], idx_v)

    @pl.when(sid == 0)
    def _():
        pltpu.sync_copy(zero, acc)

    plsc.subcore_barrier()

    def step(j, carry):
        pltpu.sync_copy(m.at[wid].at[j], m_v)
        pltpu.sync_copy(m_v, acc.at[idx_v.at[j]], add=True)
        return carry

    lax.fori_loop(0, _JS, step, 0)
    plsc.subcore_barrier()
    pltpu.sync_copy(acc.at[pl.ds(sid * _NPT, _NPT)],
                    out.at[cid].at[pl.ds(sid * _NPT, _NPT)])


_scatter = pl.kernel(
    _scatter_body,
    out_type=jax.ShapeDtypeStruct((2, _N, 32), jnp.float32),
    mesh=plsc.VectorSubcoreMesh(core_axis_name="c", subcore_axis_name="s"),
    scratch_types=[
        pltpu.VMEM((_JS, _CH), jnp.int32),
        pltpu.VMEM((_CH, 32), jnp.float32),
        pltpu.VMEM_SHARED((_N, 32), jnp.float32),
    ],
)


def _dot(a, b):
    return jnp.dot(a, b, preferred_element_type=jnp.float32)


def _enc_body(x_r, w0_r, b0_r, w1_r, b1_r, ws_r, wd_r, h_r, p_r):
    h0 = jnp.maximum(_dot(x_r[...], w0_r[...]) + b0_r[...], 0.0)
    h = jnp.maximum(_dot(h0, w1_r[...]) + b1_r[...], 0.0)
    h_r[...] = h
    p_r[0] = _dot(h, ws_r[...])
    p_r[1] = _dot(h, wd_r[...])


def _edge_body(g_r, ea_r, w0e_r, b0_r, w1_r, b1_r, w2_r, b2_r, m_r):
    t = g_r[0] + g_r[1] + _dot(ea_r[...], w0e_r[...]) + b0_r[...]
    t = jnp.maximum(t, 0.0)
    t = jnp.maximum(_dot(t, w1_r[...]) + b1_r[...], 0.0)
    m_r[...] = _dot(t, w2_r[...]) + b2_r[...]


def _groupnorm(u, gam, bet):
    gi = lax.broadcasted_iota(jnp.int32, (32, 32), 0) // 4
    gj = lax.broadcasted_iota(jnp.int32, (32, 32), 1) // 4
    avg = jnp.where(gi == gj, jnp.float32(0.25), jnp.float32(0.0))
    mu = _dot(u, avg)
    d = u - mu
    var = _dot(d * d, avg)
    return d * lax.rsqrt(var + 1e-5) * gam + bet


def _upd_common(h_r, agg_r, wa_r, wb_r, b0_r, w1_r, b1_r, w2_r, b2_r,
                gam_r, bet_r):
    agg = agg_r[0] + agg_r[1]
    u = jnp.maximum(_dot(h_r[...], wa_r[...]) + _dot(agg, wb_r[...])
                    + b0_r[...], 0.0)
    u = jnp.maximum(_dot(u, w1_r[...]) + b1_r[...], 0.0)
    u = jnp.maximum(_dot(u, w2_r[...]) + b2_r[...], 0.0)
    return _groupnorm(u, gam_r[...], bet_r[...])


def _upd_body(h_r, agg_r, wa_r, wb_r, b0_r, w1_r, b1_r, w2_r, b2_r,
              gam_r, bet_r, ws_r, wd_r, hn_r, p_r):
    hn = _upd_common(h_r, agg_r, wa_r, wb_r, b0_r, w1_r, b1_r, w2_r, b2_r,
                     gam_r, bet_r)
    hn_r[...] = hn
    p_r[0] = _dot(hn, ws_r[...])
    p_r[1] = _dot(hn, wd_r[...])


def _upd_last_body(h_r, agg_r, wa_r, wb_r, b0_r, w1_r, b1_r, w2_r, b2_r,
                   gam_r, bet_r, dw0_r, db0_r, dw1_r, db1_r, y_r):
    hn = _upd_common(h_r, agg_r, wa_r, wb_r, b0_r, w1_r, b1_r, w2_r, b2_r,
                     gam_r, bet_r)
    t = jnp.maximum(_dot(hn, dw0_r[...]) + db0_r[...], 0.0)
    y_r[...] = _dot(t, dw1_r[...]) + db1_r[...]


def _full(shape):
    return pl.BlockSpec(shape, lambda i: (0,) * len(shape))


def _enc_call(x, w0, b0, w1, b1, ws, wd):
    grid = (_N // _BN,)
    return pl.pallas_call(
        _enc_body,
        grid=grid,
        in_specs=[
            pl.BlockSpec((_BN, 128), lambda i: (i, 0)),
            _full((128, 32)), _full((1, 32)), _full((32, 32)), _full((1, 32)),
            _full((32, 32)), _full((32, 32)),
        ],
        out_specs=[
            pl.BlockSpec((_BN, 32), lambda i: (i, 0)),
            pl.BlockSpec((2, _BN, 32), lambda i: (0, i, 0)),
        ],
        out_shape=[
            jax.ShapeDtypeStruct((_N, 32), jnp.float32),
            jax.ShapeDtypeStruct((2, _N, 32), jnp.float32),
        ],
    )(x, w0, b0, w1, b1, ws, wd)


def _edge_call(g, ea, w0e, b0, w1, b1, w2, b2):
    grid = (_E // _BE,)
    return pl.pallas_call(
        _edge_body,
        grid=grid,
        in_specs=[
            pl.BlockSpec((2, _BE, 32), lambda i: (0, i, 0)),
            pl.BlockSpec((_BE, 16), lambda i: (i, 0)),
            _full((16, 32)), _full((1, 32)),
            _full((32, 32)), _full((1, 32)),
            _full((32, 32)), _full((1, 32)),
        ],
        out_specs=pl.BlockSpec((_BE, 32), lambda i: (i, 0)),
        out_shape=jax.ShapeDtypeStruct((_E, 32), jnp.float32),
    )(g, ea, w0e, b0, w1, b1, w2, b2)


def _upd_call(h, aggp, wa, wb, b0, w1, b1, w2, b2, gam, bet, ws, wd):
    grid = (_N // _BN,)
    return pl.pallas_call(
        _upd_body,
        grid=grid,
        in_specs=[
            pl.BlockSpec((_BN, 32), lambda i: (i, 0)),
            pl.BlockSpec((2, _BN, 32), lambda i: (0, i, 0)),
            _full((32, 32)), _full((32, 32)), _full((1, 32)),
            _full((32, 32)), _full((1, 32)),
            _full((32, 32)), _full((1, 32)),
            _full((1, 32)), _full((1, 32)),
            _full((32, 32)), _full((32, 32)),
        ],
        out_specs=[
            pl.BlockSpec((_BN, 32), lambda i: (i, 0)),
            pl.BlockSpec((2, _BN, 32), lambda i: (0, i, 0)),
        ],
        out_shape=[
            jax.ShapeDtypeStruct((_N, 32), jnp.float32),
            jax.ShapeDtypeStruct((2, _N, 32), jnp.float32),
        ],
    )(h, aggp, wa, wb, b0, w1, b1, w2, b2, gam, bet, ws, wd)


def _upd_last_call(h, aggp, wa, wb, b0, w1, b1, w2, b2, gam, bet,
                   dw0, db0, dw1, db1):
    grid = (_N // _BN,)
    return pl.pallas_call(
        _upd_last_body,
        grid=grid,
        in_specs=[
            pl.BlockSpec((_BN, 32), lambda i: (i, 0)),
            pl.BlockSpec((2, _BN, 32), lambda i: (0, i, 0)),
            _full((32, 32)), _full((32, 32)), _full((1, 32)),
            _full((32, 32)), _full((1, 32)),
            _full((32, 32)), _full((1, 32)),
            _full((1, 32)), _full((1, 32)),
            _full((32, 32)), _full((1, 32)),
            _full((32, 4)), _full((1, 4)),
        ],
        out_specs=pl.BlockSpec((_BN, 4), lambda i: (i, 0)),
        out_shape=jax.ShapeDtypeStruct((_N, 4), jnp.float32),
    )(h, aggp, wa, wb, b0, w1, b1, w2, b2, gam, bet, dw0, db0, dw1, db1)


def kernel(x, edge_index, edge_attr, enc_W0, enc_b0, enc_W1, enc_b1,
           msg_W0, msg_b0, msg_W1, msg_b1, msg_W2, msg_b2,
           upd_W0, upd_b0, upd_W1, upd_b1, upd_W2, upd_b2,
           gn_gamma, gn_beta, dec_W0, dec_b0, dec_W1, dec_b1):
    src = edge_index[0]
    dst = edge_index[1]
    idx2 = jnp.concatenate([src, dst + _N]).reshape(_NW, _JG, _CH)
    dstw = dst.reshape(_NW, _JS, _CH)
    zero = jnp.zeros((_N, 32), jnp.float32)

    r = lambda v: v.reshape(1, -1)
    mw_s = msg_W0[:, :32, :]
    mw_d = msg_W0[:, 32:64, :]
    mw_e = msg_W0[:, 64:, :]
    uw_a = upd_W0[:, :32, :]
    uw_b = upd_W0[:, 32:, :]
    gam = r(gn_gamma)
    bet = r(gn_beta)

    h, p = _enc_call(x, enc_W0, r(enc_b0), enc_W1, r(enc_b1),
                     mw_s[0], mw_d[0])
    for l in range(4):
        gout = _gather(p.reshape(2 * _N, 32), idx2).reshape(2, _E, 32)
        m = _edge_call(gout, edge_attr, mw_e[l], r(msg_b0[l]),
                       msg_W1[l], r(msg_b1[l]), msg_W2[l], r(msg_b2[l]))
        aggp = _scatter(m.reshape(_NW, _JS, _CH, 32), dstw, zero)
        if l < 3:
            h, p = _upd_call(h, aggp, uw_a[l], uw_b[l], r(upd_b0[l]),
                             upd_W1[l], r(upd_b1[l]), upd_W2[l], r(upd_b2[l]),
                             gam, bet, mw_s[l + 1], mw_d[l + 1])
        else:
            y = _upd_last_call(h, aggp, uw_a[l], uw_b[l], r(upd_b0[l]),
                               upd_W1[l], r(upd_b1[l]), upd_W2[l],
                               r(upd_b2[l]), gam, bet,
                               dec_W0, r(dec_b0), dec_W1, r(dec_b1))
    return y


# speedup vs baseline: 2.6508x; 2.6508x over previous
import functools

import jax
import jax.numpy as jnp
from jax import lax
from jax.experimental import pallas as pl
from jax.experimental.pallas import tpu as pltpu
from jax.experimental.pallas import tpu_sc as plsc

_N = 10000
_E = 320000
_NW = 32
_CH = 80
_JG = 2 * _E // (_NW * _CH)
_JS = _E // (_NW * _CH)
_NPT = _N // 16

_BN = 2000
_BE = 4000


def _gather_body(table, idx, out, idx_v, rows_v, sem):
    cid = lax.axis_index("c")
    sid = lax.axis_index("s")
    wid = sid * 2 + cid
    pltpu.sync_copy(idx.at[wid], idx_v)

    def step(j, carry):
        pltpu.async_copy(table.at[idx_v.at[j]], rows_v, sem).wait()
        pltpu.sync_copy(rows_v, out.at[wid].at[j])
        return carry

    lax.fori_loop(0, _JG, step, 0)


@functools.cache
def _gather_kernel():
    return pl.kernel(
        _gather_body,
        out_type=jax.ShapeDtypeStruct((_NW, _JG, _CH, 32), jnp.float32),
        mesh=plsc.VectorSubcoreMesh(core_axis_name="c", subcore_axis_name="s"),
        scratch_types=[
            pltpu.VMEM((_JG, _CH), jnp.int32),
            pltpu.VMEM((_CH, 32), jnp.float32),
            pltpu.SemaphoreType.DMA,
        ],
        compiler_params=pltpu.CompilerParams(use_tc_tiling_on_sc=False),
    )


def _gather(table, idx):
    return _gather_kernel()(table, idx)


def _scatter_body(m, dstw, zero, out, idx_v, m_v, acc):
    cid = lax.axis_index("c")
    sid = lax.axis_index("s")
    wid = sid * 2 + cid
    pltpu.sync_copy(dstw.at[wid], idx_v)

    @pl.when(sid == 0)
    def _():
        pltpu.sync_copy(zero, acc)

    plsc.subcore_barrier()

    def step(j, carry):
        pltpu.sync_copy(m.at[wid].at[j], m_v)
        pltpu.sync_copy(m_v, acc.at[idx_v.at[j]], add=True)
        return carry

    lax.fori_loop(0, _JS, step, 0)
    plsc.subcore_barrier()
    pltpu.sync_copy(acc.at[pl.ds(sid * _NPT, _NPT)],
                    out.at[cid].at[pl.ds(sid * _NPT, _NPT)])


@functools.cache
def _scatter_kernel():
    return pl.kernel(
        _scatter_body,
        out_type=jax.ShapeDtypeStruct((2, _N, 32), jnp.float32),
        mesh=plsc.VectorSubcoreMesh(core_axis_name="c", subcore_axis_name="s"),
        scratch_types=[
            pltpu.VMEM((_JS, _CH), jnp.int32),
            pltpu.VMEM((_CH, 32), jnp.float32),
            pltpu.VMEM_SHARED((_N, 32), jnp.float32),
        ],
        compiler_params=pltpu.CompilerParams(use_tc_tiling_on_sc=False),
    )


def _scatter(m, dstw, zero):
    return _scatter_kernel()(m, dstw, zero)


def _dot(a, b):
    return jnp.dot(a, b, preferred_element_type=jnp.float32)


def _enc_body(x_r, w0_r, b0_r, w1_r, b1_r, ws_r, wd_r, h_r, p_r):
    h0 = jnp.maximum(_dot(x_r[...], w0_r[...]) + b0_r[...], 0.0)
    h = jnp.maximum(_dot(h0, w1_r[...]) + b1_r[...], 0.0)
    h_r[...] = h
    p_r[0] = _dot(h, ws_r[...])
    p_r[1] = _dot(h, wd_r[...])


def _edge_body(g_r, ea_r, w0e_r, b0_r, w1_r, b1_r, w2_r, b2_r, m_r):
    t = g_r[0] + g_r[1] + _dot(ea_r[...], w0e_r[...]) + b0_r[...]
    t = jnp.maximum(t, 0.0)
    t = jnp.maximum(_dot(t, w1_r[...]) + b1_r[...], 0.0)
    m_r[...] = _dot(t, w2_r[...]) + b2_r[...]


def _gsum4(v):
    lane = lax.broadcasted_iota(jnp.int32, v.shape, len(v.shape) - 1)
    s1 = v + jnp.where(lane % 2 == 0, pltpu.roll(v, 31, len(v.shape) - 1),
                       pltpu.roll(v, 1, len(v.shape) - 1))
    s2 = s1 + jnp.where(lane % 4 < 2, pltpu.roll(s1, 30, len(v.shape) - 1),
                        pltpu.roll(s1, 2, len(v.shape) - 1))
    return s2


def _groupnorm(u, gam, bet):
    mu = _gsum4(u) * 0.25
    d = u - mu
    var = _gsum4(d * d) * 0.25
    return d * lax.rsqrt(var + 1e-5) * gam + bet


def _upd_common(h_r, agg_r, wa_r, wb_r, b0_r, w1_r, b1_r, w2_r, b2_r,
                gam_r, bet_r):
    agg = agg_r[0] + agg_r[1]
    u = jnp.maximum(_dot(h_r[...], wa_r[...]) + _dot(agg, wb_r[...])
                    + b0_r[...], 0.0)
    u = jnp.maximum(_dot(u, w1_r[...]) + b1_r[...], 0.0)
    u = jnp.maximum(_dot(u, w2_r[...]) + b2_r[...], 0.0)
    return _groupnorm(u, gam_r[...], bet_r[...])


def _upd_body(h_r, agg_r, wa_r, wb_r, b0_r, w1_r, b1_r, w2_r, b2_r,
              gam_r, bet_r, ws_r, wd_r, hn_r, p_r):
    hn = _upd_common(h_r, agg_r, wa_r, wb_r, b0_r, w1_r, b1_r, w2_r, b2_r,
                     gam_r, bet_r)
    hn_r[...] = hn
    p_r[0] = _dot(hn, ws_r[...])
    p_r[1] = _dot(hn, wd_r[...])


def _upd_last_body(h_r, agg_r, wa_r, wb_r, b0_r, w1_r, b1_r, w2_r, b2_r,
                   gam_r, bet_r, dw0_r, db0_r, dw1_r, db1_r, y_r):
    hn = _upd_common(h_r, agg_r, wa_r, wb_r, b0_r, w1_r, b1_r, w2_r, b2_r,
                     gam_r, bet_r)
    t = jnp.maximum(_dot(hn, dw0_r[...]) + db0_r[...], 0.0)
    y_r[...] = _dot(t, dw1_r[...]) + db1_r[...]


def _full(shape):
    return pl.BlockSpec(shape, lambda i: (0,) * len(shape))


def _enc_call(x, w0, b0, w1, b1, ws, wd):
    grid = (_N // _BN,)
    return pl.pallas_call(
        _enc_body,
        grid=grid,
        in_specs=[
            pl.BlockSpec((_BN, 128), lambda i: (i, 0)),
            _full((128, 32)), _full((1, 32)), _full((32, 32)), _full((1, 32)),
            _full((32, 32)), _full((32, 32)),
        ],
        out_specs=[
            pl.BlockSpec((_BN, 32), lambda i: (i, 0)),
            pl.BlockSpec((2, _BN, 32), lambda i: (0, i, 0)),
        ],
        out_shape=[
            jax.ShapeDtypeStruct((_N, 32), jnp.float32),
            jax.ShapeDtypeStruct((2, _N, 32), jnp.float32),
        ],
    )(x, w0, b0, w1, b1, ws, wd)


def _edge_call(g, ea, w0e, b0, w1, b1, w2, b2):
    grid = (_E // _BE,)
    return pl.pallas_call(
        _edge_body,
        grid=grid,
        in_specs=[
            pl.BlockSpec((2, _BE, 32), lambda i: (0, i, 0)),
            pl.BlockSpec((_BE, 16), lambda i: (i, 0)),
            _full((16, 32)), _full((1, 32)),
            _full((32, 32)), _full((1, 32)),
            _full((32, 32)), _full((1, 32)),
        ],
        out_specs=pl.BlockSpec((_BE, 32), lambda i: (i, 0)),
        out_shape=jax.ShapeDtypeStruct((_E, 32), jnp.float32),
    )(g, ea, w0e, b0, w1, b1, w2, b2)


def _upd_call(h, aggp, wa, wb, b0, w1, b1, w2, b2, gam, bet, ws, wd):
    grid = (_N // _BN,)
    return pl.pallas_call(
        _upd_body,
        grid=grid,
        in_specs=[
            pl.BlockSpec((_BN, 32), lambda i: (i, 0)),
            pl.BlockSpec((2, _BN, 32), lambda i: (0, i, 0)),
            _full((32, 32)), _full((32, 32)), _full((1, 32)),
            _full((32, 32)), _full((1, 32)),
            _full((32, 32)), _full((1, 32)),
            _full((1, 32)), _full((1, 32)),
            _full((32, 32)), _full((32, 32)),
        ],
        out_specs=[
            pl.BlockSpec((_BN, 32), lambda i: (i, 0)),
            pl.BlockSpec((2, _BN, 32), lambda i: (0, i, 0)),
        ],
        out_shape=[
            jax.ShapeDtypeStruct((_N, 32), jnp.float32),
            jax.ShapeDtypeStruct((2, _N, 32), jnp.float32),
        ],
    )(h, aggp, wa, wb, b0, w1, b1, w2, b2, gam, bet, ws, wd)


def _upd_last_call(h, aggp, wa, wb, b0, w1, b1, w2, b2, gam, bet,
                   dw0, db0, dw1, db1):
    grid = (_N // _BN,)
    return pl.pallas_call(
        _upd_last_body,
        grid=grid,
        in_specs=[
            pl.BlockSpec((_BN, 32), lambda i: (i, 0)),
            pl.BlockSpec((2, _BN, 32), lambda i: (0, i, 0)),
            _full((32, 32)), _full((32, 32)), _full((1, 32)),
            _full((32, 32)), _full((1, 32)),
            _full((32, 32)), _full((1, 32)),
            _full((1, 32)), _full((1, 32)),
            _full((32, 32)), _full((1, 32)),
            _full((32, 4)), _full((1, 4)),
        ],
        out_specs=pl.BlockSpec((_BN, 4), lambda i: (i, 0)),
        out_shape=jax.ShapeDtypeStruct((_N, 4), jnp.float32),
    )(h, aggp, wa, wb, b0, w1, b1, w2, b2, gam, bet, dw0, db0, dw1, db1)


def kernel(x, edge_index, edge_attr, enc_W0, enc_b0, enc_W1, enc_b1,
           msg_W0, msg_b0, msg_W1, msg_b1, msg_W2, msg_b2,
           upd_W0, upd_b0, upd_W1, upd_b1, upd_W2, upd_b2,
           gn_gamma, gn_beta, dec_W0, dec_b0, dec_W1, dec_b1):
    src = edge_index[0]
    dst = edge_index[1]
    idx2 = jnp.concatenate([src, dst + _N]).reshape(_NW, _JG, _CH)
    dstw = dst.reshape(_NW, _JS, _CH)
    zero = jnp.zeros((_N, 32), jnp.float32)

    r = lambda v: v.reshape(1, -1)
    mw_s = msg_W0[:, :32, :]
    mw_d = msg_W0[:, 32:64, :]
    mw_e = msg_W0[:, 64:, :]
    uw_a = upd_W0[:, :32, :]
    uw_b = upd_W0[:, 32:, :]
    gam = r(gn_gamma)
    bet = r(gn_beta)

    h, p = _enc_call(x, enc_W0, r(enc_b0), enc_W1, r(enc_b1),
                     mw_s[0], mw_d[0])
    for l in range(4):
        gout = _gather(p.reshape(2 * _N, 32), idx2).reshape(2, _E, 32)
        m = _edge_call(gout, edge_attr, mw_e[l], r(msg_b0[l]),
                       msg_W1[l], r(msg_b1[l]), msg_W2[l], r(msg_b2[l]))
        aggp = _scatter(m.reshape(_NW, _JS, _CH, 32), dstw, zero)
        if l < 3:
            h, p = _upd_call(h, aggp, uw_a[l], uw_b[l], r(upd_b0[l]),
                             upd_W1[l], r(upd_b1[l]), upd_W2[l], r(upd_b2[l]),
                             gam, bet, mw_s[l + 1], mw_d[l + 1])
        else:
            y = _upd_last_call(h, aggp, uw_a[l], uw_b[l], r(upd_b0[l]),
                               upd_W1[l], r(upd_b1[l]), upd_W2[l],
                               r(upd_b2[l]), gam, bet,
                               dec_W0, r(dec_b0), dec_W1, r(dec_b1))
    return y


# speedup vs baseline: 3.4672x; 1.3080x over previous
import functools

import jax
import jax.numpy as jnp
from jax import lax
from jax.experimental import pallas as pl
from jax.experimental.pallas import tpu as pltpu
from jax.experimental.pallas import tpu_sc as plsc

_N = 10000
_E = 320000
_NW = 32
_CH = 100
_K = 5
_JG = 2 * _E // (_NW * _CH)
_JS = _E // (_NW * _CH)
_NPT = _N // 16

_BN = 2000
_BE = 4000


def _gather_body(table, idx, out, idx_v, rows0, rows1, gsem, osem0, osem1):
    cid = lax.axis_index("c")
    sid = lax.axis_index("s")
    wid = sid * 2 + cid
    pltpu.sync_copy(idx.at[wid], idx_v)

    rows = (rows0, rows1)
    osems = (osem0, osem1)

    def process(b, g):
        @pl.when(b >= 2)
        def _():
            pltpu.make_async_copy(
                rows[g], out.at[wid].at[pl.ds(0, _K)], osems[g]).wait()
        for k in range(_K):
            pltpu.async_copy(table.at[idx_v.at[b * _K + k]],
                             rows[g].at[k], gsem)
        for k in range(_K):
            pltpu.make_async_copy(table.at[idx_v.at[0]],
                                  rows[g].at[k], gsem).wait()
        pltpu.async_copy(rows[g], out.at[wid].at[pl.ds(b * _K, _K)], osems[g])

    def pair(t, carry):
        process(2 * t, 0)
        process(2 * t + 1, 1)
        return carry

    lax.fori_loop(0, _JG // (2 * _K), pair, 0)
    for g in (0, 1):
        pltpu.make_async_copy(rows[g], out.at[wid].at[pl.ds(0, _K)],
                              osems[g]).wait()


@functools.cache
def _gather_kernel():
    return pl.kernel(
        _gather_body,
        out_type=jax.ShapeDtypeStruct((_NW, _JG, _CH, 32), jnp.float32),
        mesh=plsc.VectorSubcoreMesh(core_axis_name="c", subcore_axis_name="s"),
        scratch_types=[
            pltpu.VMEM((_JG, _CH), jnp.int32),
            pltpu.VMEM((_K, _CH, 32), jnp.float32),
            pltpu.VMEM((_K, _CH, 32), jnp.float32),
            pltpu.SemaphoreType.DMA,
            pltpu.SemaphoreType.DMA,
            pltpu.SemaphoreType.DMA,
        ],
        compiler_params=pltpu.CompilerParams(use_tc_tiling_on_sc=False),
    )


def _gather(table, idx):
    return _gather_kernel()(table, idx)


def _scatter_body(m, dstw, zero, out, idx_v, m0, m1, acc,
                  msem0, msem1, asem0, asem1):
    cid = lax.axis_index("c")
    sid = lax.axis_index("s")
    wid = sid * 2 + cid
    pltpu.sync_copy(dstw.at[wid], idx_v)

    @pl.when(sid == 0)
    def _():
        pltpu.sync_copy(zero, acc)

    plsc.subcore_barrier()

    mbuf = (m0, m1)
    msems = (msem0, msem1)
    asems = (asem0, asem1)

    def process(b, g):
        @pl.when(b >= 2)
        def _():
            for k in range(_K):
                pltpu.make_async_copy(mbuf[g].at[k], acc.at[idx_v.at[0]],
                                      asems[g]).wait()
        pltpu.async_copy(m.at[wid].at[pl.ds(b * _K, _K)], mbuf[g], msems[g])
        pltpu.make_async_copy(m.at[wid].at[pl.ds(0, _K)], mbuf[g],
                              msems[g]).wait()
        for k in range(_K):
            pltpu.async_copy(mbuf[g].at[k], acc.at[idx_v.at[b * _K + k]],
                             asems[g], add=True)

    def pair(t, carry):
        process(2 * t, 0)
        process(2 * t + 1, 1)
        return carry

    lax.fori_loop(0, _JS // (2 * _K), pair, 0)
    for g in (0, 1):
        for k in range(_K):
            pltpu.make_async_copy(mbuf[g].at[k], acc.at[idx_v.at[0]],
                                  asems[g]).wait()
    plsc.subcore_barrier()
    pltpu.sync_copy(acc.at[pl.ds(sid * _NPT, _NPT)],
                    out.at[cid].at[pl.ds(sid * _NPT, _NPT)])


@functools.cache
def _scatter_kernel():
    return pl.kernel(
        _scatter_body,
        out_type=jax.ShapeDtypeStruct((2, _N, 32), jnp.float32),
        mesh=plsc.VectorSubcoreMesh(core_axis_name="c", subcore_axis_name="s"),
        scratch_types=[
            pltpu.VMEM((_JS, _CH), jnp.int32),
            pltpu.VMEM((_K, _CH, 32), jnp.float32),
            pltpu.VMEM((_K, _CH, 32), jnp.float32),
            pltpu.VMEM_SHARED((_N, 32), jnp.float32),
            pltpu.SemaphoreType.DMA,
            pltpu.SemaphoreType.DMA,
            pltpu.SemaphoreType.DMA,
            pltpu.SemaphoreType.DMA,
        ],
        compiler_params=pltpu.CompilerParams(use_tc_tiling_on_sc=False),
    )


def _scatter(m, dstw, zero):
    return _scatter_kernel()(m, dstw, zero)


def _dot(a, b):
    return jnp.dot(a, b, preferred_element_type=jnp.float32)


def _enc_body(x_r, w0_r, b0_r, w1_r, b1_r, ws_r, wd_r, h_r, p_r):
    h0 = jnp.maximum(_dot(x_r[...], w0_r[...]) + b0_r[...], 0.0)
    h = jnp.maximum(_dot(h0, w1_r[...]) + b1_r[...], 0.0)
    h_r[...] = h
    p_r[0] = _dot(h, ws_r[...])
    p_r[1] = _dot(h, wd_r[...])


def _edge_body(g_r, ea_r, w0e_r, b0_r, w1_r, b1_r, w2_r, b2_r, m_r):
    t = g_r[0] + g_r[1] + _dot(ea_r[...], w0e_r[...]) + b0_r[...]
    t = jnp.maximum(t, 0.0)
    t = jnp.maximum(_dot(t, w1_r[...]) + b1_r[...], 0.0)
    m_r[...] = _dot(t, w2_r[...]) + b2_r[...]


def _gsum4(v):
    lane = lax.broadcasted_iota(jnp.int32, v.shape, len(v.shape) - 1)
    s1 = v + jnp.where(lane % 2 == 0, pltpu.roll(v, 31, len(v.shape) - 1),
                       pltpu.roll(v, 1, len(v.shape) - 1))
    s2 = s1 + jnp.where(lane % 4 < 2, pltpu.roll(s1, 30, len(v.shape) - 1),
                        pltpu.roll(s1, 2, len(v.shape) - 1))
    return s2


def _groupnorm(u, gam, bet):
    mu = _gsum4(u) * 0.25
    d = u - mu
    var = _gsum4(d * d) * 0.25
    return d * lax.rsqrt(var + 1e-5) * gam + bet


def _upd_common(h_r, agg_r, wa_r, wb_r, b0_r, w1_r, b1_r, w2_r, b2_r,
                gam_r, bet_r):
    agg = agg_r[0] + agg_r[1]
    u = jnp.maximum(_dot(h_r[...], wa_r[...]) + _dot(agg, wb_r[...])
                    + b0_r[...], 0.0)
    u = jnp.maximum(_dot(u, w1_r[...]) + b1_r[...], 0.0)
    u = jnp.maximum(_dot(u, w2_r[...]) + b2_r[...], 0.0)
    return _groupnorm(u, gam_r[...], bet_r[...])


def _upd_body(h_r, agg_r, wa_r, wb_r, b0_r, w1_r, b1_r, w2_r, b2_r,
              gam_r, bet_r, ws_r, wd_r, hn_r, p_r):
    hn = _upd_common(h_r, agg_r, wa_r, wb_r, b0_r, w1_r, b1_r, w2_r, b2_r,
                     gam_r, bet_r)
    hn_r[...] = hn
    p_r[0] = _dot(hn, ws_r[...])
    p_r[1] = _dot(hn, wd_r[...])


def _upd_last_body(h_r, agg_r, wa_r, wb_r, b0_r, w1_r, b1_r, w2_r, b2_r,
                   gam_r, bet_r, dw0_r, db0_r, dw1_r, db1_r, y_r):
    hn = _upd_common(h_r, agg_r, wa_r, wb_r, b0_r, w1_r, b1_r, w2_r, b2_r,
                     gam_r, bet_r)
    t = jnp.maximum(_dot(hn, dw0_r[...]) + db0_r[...], 0.0)
    y_r[...] = _dot(t, dw1_r[...]) + db1_r[...]


def _full(shape):
    return pl.BlockSpec(shape, lambda i: (0,) * len(shape))


def _enc_call(x, w0, b0, w1, b1, ws, wd):
    grid = (_N // _BN,)
    return pl.pallas_call(
        _enc_body,
        grid=grid,
        in_specs=[
            pl.BlockSpec((_BN, 128), lambda i: (i, 0)),
            _full((128, 32)), _full((1, 32)), _full((32, 32)), _full((1, 32)),
            _full((32, 32)), _full((32, 32)),
        ],
        out_specs=[
            pl.BlockSpec((_BN, 32), lambda i: (i, 0)),
            pl.BlockSpec((2, _BN, 32), lambda i: (0, i, 0)),
        ],
        out_shape=[
            jax.ShapeDtypeStruct((_N, 32), jnp.float32),
            jax.ShapeDtypeStruct((2, _N, 32), jnp.float32),
        ],
    )(x, w0, b0, w1, b1, ws, wd)


def _edge_call(g, ea, w0e, b0, w1, b1, w2, b2):
    grid = (_E // _BE,)
    return pl.pallas_call(
        _edge_body,
        grid=grid,
        in_specs=[
            pl.BlockSpec((2, _BE, 32), lambda i: (0, i, 0)),
            pl.BlockSpec((_BE, 16), lambda i: (i, 0)),
            _full((16, 32)), _full((1, 32)),
            _full((32, 32)), _full((1, 32)),
            _full((32, 32)), _full((1, 32)),
        ],
        out_specs=pl.BlockSpec((_BE, 32), lambda i: (i, 0)),
        out_shape=jax.ShapeDtypeStruct((_E, 32), jnp.float32),
    )(g, ea, w0e, b0, w1, b1, w2, b2)


def _upd_call(h, aggp, wa, wb, b0, w1, b1, w2, b2, gam, bet, ws, wd):
    grid = (_N // _BN,)
    return pl.pallas_call(
        _upd_body,
        grid=grid,
        in_specs=[
            pl.BlockSpec((_BN, 32), lambda i: (i, 0)),
            pl.BlockSpec((2, _BN, 32), lambda i: (0, i, 0)),
            _full((32, 32)), _full((32, 32)), _full((1, 32)),
            _full((32, 32)), _full((1, 32)),
            _full((32, 32)), _full((1, 32)),
            _full((1, 32)), _full((1, 32)),
            _full((32, 32)), _full((32, 32)),
        ],
        out_specs=[
            pl.BlockSpec((_BN, 32), lambda i: (i, 0)),
            pl.BlockSpec((2, _BN, 32), lambda i: (0, i, 0)),
        ],
        out_shape=[
            jax.ShapeDtypeStruct((_N, 32), jnp.float32),
            jax.ShapeDtypeStruct((2, _N, 32), jnp.float32),
        ],
    )(h, aggp, wa, wb, b0, w1, b1, w2, b2, gam, bet, ws, wd)


def _upd_last_call(h, aggp, wa, wb, b0, w1, b1, w2, b2, gam, bet,
                   dw0, db0, dw1, db1):
    grid = (_N // _BN,)
    return pl.pallas_call(
        _upd_last_body,
        grid=grid,
        in_specs=[
            pl.BlockSpec((_BN, 32), lambda i: (i, 0)),
            pl.BlockSpec((2, _BN, 32), lambda i: (0, i, 0)),
            _full((32, 32)), _full((32, 32)), _full((1, 32)),
            _full((32, 32)), _full((1, 32)),
            _full((32, 32)), _full((1, 32)),
            _full((1, 32)), _full((1, 32)),
            _full((32, 32)), _full((1, 32)),
            _full((32, 4)), _full((1, 4)),
        ],
        out_specs=pl.BlockSpec((_BN, 4), lambda i: (i, 0)),
        out_shape=jax.ShapeDtypeStruct((_N, 4), jnp.float32),
    )(h, aggp, wa, wb, b0, w1, b1, w2, b2, gam, bet, dw0, db0, dw1, db1)


def kernel(x, edge_index, edge_attr, enc_W0, enc_b0, enc_W1, enc_b1,
           msg_W0, msg_b0, msg_W1, msg_b1, msg_W2, msg_b2,
           upd_W0, upd_b0, upd_W1, upd_b1, upd_W2, upd_b2,
           gn_gamma, gn_beta, dec_W0, dec_b0, dec_W1, dec_b1):
    src = edge_index[0]
    dst = edge_index[1]
    idx2 = jnp.concatenate([src, dst + _N]).reshape(_NW, _JG, _CH)
    dstw = dst.reshape(_NW, _JS, _CH)
    zero = jnp.zeros((_N, 32), jnp.float32)

    r = lambda v: v.reshape(1, -1)
    mw_s = msg_W0[:, :32, :]
    mw_d = msg_W0[:, 32:64, :]
    mw_e = msg_W0[:, 64:, :]
    uw_a = upd_W0[:, :32, :]
    uw_b = upd_W0[:, 32:, :]
    gam = r(gn_gamma)
    bet = r(gn_beta)

    h, p = _enc_call(x, enc_W0, r(enc_b0), enc_W1, r(enc_b1),
                     mw_s[0], mw_d[0])
    for l in range(4):
        gout = _gather(p.reshape(2 * _N, 32), idx2).reshape(2, _E, 32)
        m = _edge_call(gout, edge_attr, mw_e[l], r(msg_b0[l]),
                       msg_W1[l], r(msg_b1[l]), msg_W2[l], r(msg_b2[l]))
        aggp = _scatter(m.reshape(_NW, _JS, _CH, 32), dstw, zero)
        if l < 3:
            h, p = _upd_call(h, aggp, uw_a[l], uw_b[l], r(upd_b0[l]),
                             upd_W1[l], r(upd_b1[l]), upd_W2[l], r(upd_b2[l]),
                             gam, bet, mw_s[l + 1], mw_d[l + 1])
        else:
            y = _upd_last_call(h, aggp, uw_a[l], uw_b[l], r(upd_b0[l]),
                               upd_W1[l], r(upd_b1[l]), upd_W2[l],
                               r(upd_b2[l]), gam, bet,
                               dec_W0, r(dec_b0), dec_W1, r(dec_b1))
    return y
